# baseline (speedup 1.0000x reference)
import functools

import jax
import jax.numpy as jnp
from jax import lax
from jax.experimental import pallas as pl
from jax.experimental.pallas import tpu as pltpu
from jax.experimental.pallas import tpu_sc as plsc

VOCAB = 100000
EMBED = 64
SEQ = 20
HIDDEN = 256
BATCH = 1024


_NC, _NS = 2, 16
_NW = _NC * _NS

_ROWS = BATCH * SEQ
_B_PER_W = _ROWS // _NW


def _sc_gather(table, idx):
    @functools.partial(
        pl.kernel,
        mesh=plsc.VectorSubcoreMesh(core_axis_name="c", subcore_axis_name="s"),
        out_type=jax.ShapeDtypeStruct((_ROWS, EMBED), jnp.float32),
        scratch_types=[
            pltpu.VMEM((_B_PER_W,), jnp.int32),
            pltpu.VMEM((_B_PER_W, EMBED), jnp.float32),
            pltpu.SemaphoreType.DMA,
        ],
        compiler_params=pltpu.CompilerParams(use_tc_tiling_on_sc=False),
    )
    def k(table_hbm, idx_hbm, out_hbm, idx_v, rows_v, sem):
        wid = lax.axis_index("s") * _NC + lax.axis_index("c")
        base = wid * _B_PER_W
        pltpu.sync_copy(idx_hbm.at[pl.ds(base, _B_PER_W)], idx_v)
        pltpu.async_copy(table_hbm.at[idx_v], rows_v, sem).wait()
        pltpu.sync_copy(rows_v, out_hbm.at[pl.ds(base, _B_PER_W)])

    return k(table, idx)


_TV = 2048
_GRID = (VOCAB + _TV - 1) // _TV


def _mlp_body(flat_ref, w2_ref, b2_ref, w1_ref, b1_ref, w3_ref, out_ref, y2_scr):
    @pl.when(pl.program_id(0) == 0)
    def _():
        h = lax.dot_general(
            flat_ref[...], w2_ref[...], (((1,), (1,)), ((), ())),
            preferred_element_type=jnp.float32)
        y2_scr[...] = jnp.tanh(h + b2_ref[...])

    y1 = lax.dot_general(
        flat_ref[...].astype(jnp.bfloat16), w1_ref[...].astype(jnp.bfloat16),
        (((1,), (1,)), ((), ())),
        preferred_element_type=jnp.float32)
    y3 = lax.dot_general(
        y2_scr[...].astype(jnp.bfloat16), w3_ref[...].astype(jnp.bfloat16),
        (((1,), (1,)), ((), ())),
        preferred_element_type=jnp.float32)
    out_ref[...] = y1 + y3 + b1_ref[...]


def _mlp(flat, W1, b1, W2, b2, W3):
    in_dim = SEQ * EMBED
    return pl.pallas_call(
        _mlp_body,
        grid=(_GRID,),
        in_specs=[
            pl.BlockSpec((BATCH, in_dim), lambda i: (0, 0)),
            pl.BlockSpec((HIDDEN, in_dim), lambda i: (0, 0)),
            pl.BlockSpec((1, HIDDEN), lambda i: (0, 0)),
            pl.BlockSpec((_TV, in_dim), lambda i: (i, 0)),
            pl.BlockSpec((1, _TV), lambda i: (0, i)),
            pl.BlockSpec((_TV, HIDDEN), lambda i: (i, 0)),
        ],
        out_specs=pl.BlockSpec((BATCH, _TV), lambda i: (0, i)),
        out_shape=jax.ShapeDtypeStruct((BATCH, VOCAB), jnp.float32),
        scratch_shapes=[pltpu.VMEM((BATCH, HIDDEN), jnp.float32)],
    )(flat, W2, b2.reshape(1, HIDDEN), W1, b1.reshape(1, VOCAB), W3)


def kernel(x, emb, W1, b1, W2, b2, W3):
    rows = _sc_gather(emb, x.reshape(-1))
    flat = rows.reshape(BATCH, SEQ * EMBED)
    return _mlp(flat, W1, b1, W2, b2, W3)


# speedup vs baseline: 1.6116x; 1.6116x over previous
import functools

import jax
import jax.numpy as jnp
from jax import lax
from jax.experimental import pallas as pl
from jax.experimental.pallas import tpu as pltpu
from jax.experimental.pallas import tpu_sc as plsc

VOCAB = 100000
EMBED = 64
SEQ = 20
HIDDEN = 256
BATCH = 1024


_NC, _NS = 2, 16
_NW = _NC * _NS

_ROWS = BATCH * SEQ
_B_PER_W = _ROWS // _NW


def _sc_gather(table, idx):
    @functools.partial(
        pl.kernel,
        mesh=plsc.VectorSubcoreMesh(core_axis_name="c", subcore_axis_name="s"),
        out_type=jax.ShapeDtypeStruct((_ROWS, EMBED), jnp.float32),
        scratch_types=[
            pltpu.VMEM((_B_PER_W,), jnp.int32),
            pltpu.VMEM((_B_PER_W, EMBED), jnp.float32),
            pltpu.SemaphoreType.DMA,
        ],
        compiler_params=pltpu.CompilerParams(use_tc_tiling_on_sc=False),
    )
    def k(table_hbm, idx_hbm, out_hbm, idx_v, rows_v, sem):
        wid = lax.axis_index("s") * _NC + lax.axis_index("c")
        base = wid * _B_PER_W
        pltpu.sync_copy(idx_hbm.at[pl.ds(base, _B_PER_W)], idx_v)
        pltpu.async_copy(table_hbm.at[idx_v], rows_v, sem).wait()
        pltpu.sync_copy(rows_v, out_hbm.at[pl.ds(base, _B_PER_W)])

    return k(table, idx)


_TV = 2048
_GRID = (VOCAB + _TV - 1) // _TV


def _mlp_body(flat_ref, w2_ref, b2_ref, w1_ref, b1_ref, w3_ref, out_ref, y2_scr):
    @pl.when(pl.program_id(0) == 0)
    def _():
        h = lax.dot_general(
            flat_ref[...], w2_ref[...], (((1,), (1,)), ((), ())),
            preferred_element_type=jnp.float32)
        y2_scr[...] = jnp.tanh(h + b2_ref[...])

    y1 = lax.dot_general(
        w1_ref[...], flat_ref[...], (((1,), (1,)), ((), ())),
        preferred_element_type=jnp.float32)
    y3 = lax.dot_general(
        w3_ref[...], y2_scr[...], (((1,), (1,)), ((), ())),
        preferred_element_type=jnp.float32)
    out_ref[...] = y1 + y3 + b1_ref[...]


def _mlp(flat, W1, b1, W2, b2, W3):
    in_dim = SEQ * EMBED
    out_t = pl.pallas_call(
        _mlp_body,
        grid=(_GRID,),
        in_specs=[
            pl.BlockSpec((BATCH, in_dim), lambda i: (0, 0)),
            pl.BlockSpec((HIDDEN, in_dim), lambda i: (0, 0)),
            pl.BlockSpec((1, HIDDEN), lambda i: (0, 0)),
            pl.BlockSpec((_TV, in_dim), lambda i: (i, 0)),
            pl.BlockSpec((_TV, 1), lambda i: (i, 0)),
            pl.BlockSpec((_TV, HIDDEN), lambda i: (i, 0)),
        ],
        out_specs=pl.BlockSpec((_TV, BATCH), lambda i: (i, 0)),
        out_shape=jax.ShapeDtypeStruct((VOCAB, BATCH), jnp.float32),
        scratch_shapes=[pltpu.VMEM((BATCH, HIDDEN), jnp.float32)],
        compiler_params=pltpu.CompilerParams(vmem_limit_bytes=100 * 1024 * 1024),
    )(flat, W2, b2.reshape(1, HIDDEN), W1, b1.reshape(VOCAB, 1), W3)
    return out_t.T


def kernel(x, emb, W1, b1, W2, b2, W3):
    rows = _sc_gather(emb, x.reshape(-1))
    flat = rows.reshape(BATCH, SEQ * EMBED)
    return _mlp(flat, W1, b1, W2, b2, W3)
